# baseline (speedup 1.0000x reference)
import jax
import jax.numpy as jnp
from jax import lax
from jax.experimental import pallas as pl
from jax.experimental.pallas import tpu as pltpu

_N = 16384
_BLK = 4096
_NBLK = _N // _BLK
_KNN_R = 8
_TOPK = 3
_NEIGHK = 24
_THRESH = 0.5
_NEG = -1e30


def _branch1_body(x_ref, mW_ref, mb_ref, aW_ref, ab_ref, gW_ref, gb_ref,
                  cW_ref, cb_ref, kW_ref, kb_ref,
                  ip_ref, m1_ref, l1_ref, t3_ref,
                  stat_ref, macc_ref, t3v_ref, t3i_ref):
    i = pl.program_id(0)

    @pl.when(i == 0)
    def _init():
        stat_ref[0] = _NEG
        stat_ref[1] = 0.0
        macc_ref[...] = jnp.zeros_like(macc_ref)
        t3v_ref[0] = _NEG
        t3v_ref[1] = _NEG
        t3v_ref[2] = _NEG
        t3i_ref[0] = 0
        t3i_ref[1] = 0
        t3i_ref[2] = 0

    x = x_ref[...]
    h = jnp.maximum(
        jnp.dot(x, mW_ref[...], preferred_element_type=jnp.float32)
        + mb_ref[...], 0.0)
    a = jnp.tanh(
        jnp.dot(h, aW_ref[...], preferred_element_type=jnp.float32)
        + ab_ref[...])
    g = jax.nn.sigmoid(
        jnp.dot(h, gW_ref[...], preferred_element_type=jnp.float32)
        + gb_ref[...])
    A = jnp.sum(a * g * cW_ref[...], axis=1, keepdims=True) + cb_ref[0, 0]
    il = jnp.dot(h, kW_ref[...], preferred_element_type=jnp.float32) + kb_ref[...]
    ilm = jnp.max(il, axis=1, keepdims=True)
    pe = jnp.exp(il - ilm)
    p1 = pe[:, 1:2] / (pe[:, 0:1] + pe[:, 1:2])
    probs = jax.nn.sigmoid(A) * p1
    ip_ref[...] = probs

    bm = jnp.max(A)
    m_old = stat_ref[0]
    d_old = stat_ref[1]
    m_new = jnp.maximum(m_old, bm)
    alpha = jnp.exp(m_old - m_new)
    w = jnp.exp(A - m_new)
    stat_ref[0] = m_new
    stat_ref[1] = d_old * alpha + jnp.sum(w)
    contrib = lax.dot_general(w, h, (((0,), (0,)), ((), ())),
                              preferred_element_type=jnp.float32)
    macc_ref[...] = macc_ref[...] * alpha + contrib

    gidx = i * _BLK + lax.broadcasted_iota(jnp.int32, (_BLK, 1), 0)
    pv = probs
    for _ in range(_TOPK):
        bv = jnp.max(pv)
        bi = jnp.min(jnp.where(pv == bv, gidx, _N))
        v0, v1, v2 = t3v_ref[0], t3v_ref[1], t3v_ref[2]
        i0, i1, i2 = t3i_ref[0], t3i_ref[1], t3i_ref[2]
        gt0 = bv > v0
        gt1 = bv > v1
        gt2 = bv > v2
        t3v_ref[0] = jnp.where(gt0, bv, v0)
        t3i_ref[0] = jnp.where(gt0, bi, i0)
        t3v_ref[1] = jnp.where(gt0, v0, jnp.where(gt1, bv, v1))
        t3i_ref[1] = jnp.where(gt0, i0, jnp.where(gt1, bi, i1))
        t3v_ref[2] = jnp.where(gt0, v1, jnp.where(gt1, v1, jnp.where(gt2, bv, v2)))
        t3i_ref[2] = jnp.where(gt0, i1, jnp.where(gt1, i1, jnp.where(gt2, bi, i2)))
        pv = jnp.where(gidx == bi, _NEG, pv)

    @pl.when(i == _NBLK - 1)
    def _fin():
        M1 = macc_ref[...] / stat_ref[1]
        m1_ref[...] = M1
        l1_ref[...] = jnp.dot(M1, kW_ref[...],
                              preferred_element_type=jnp.float32) + kb_ref[...]
        lanes = lax.broadcasted_iota(jnp.int32, (1, 8), 1)
        t0, t1, t2 = t3i_ref[0], t3i_ref[1], t3i_ref[2]
        t3_ref[...] = jnp.where(lanes == 0, t0,
                                jnp.where(lanes == 1, t1,
                                          jnp.where(lanes == 2, t2, 0)))


def _knn_body(t3_ref, ip_ref, c1_ref, c2x_ref, c2y_ref, qm_ref):
    qm_ref[...] = jnp.zeros_like(qm_ref)
    c2x = c2x_ref[...]
    c2y = c2y_ref[...]
    lanes = lax.broadcasted_iota(jnp.int32, (_KNN_R, _N), 1)

    def heavy(rows8, valid):
        rx = rows8[:, 0:1]
        ry = rows8[:, 1:2]
        dx = rx - c2x
        dy = ry - c2y
        d2 = dx * dx + dy * dy
        work = d2
        m = jnp.zeros((_KNN_R, 1), jnp.float32)
        am = jnp.zeros((_KNN_R, 1), jnp.int32)
        for t in range(_NEIGHK):
            m = jnp.min(work, axis=1, keepdims=True)
            am = jnp.min(jnp.where(work == m, lanes, _N), axis=1, keepdims=True)
            if t < _NEIGHK - 1:
                work = jnp.where(lanes == am, jnp.inf, work)
        member = (d2 < m) | ((d2 == m) & (lanes <= am))
        if valid is not None:
            member = member & valid
        hit = jnp.max(member.astype(jnp.float32), axis=0, keepdims=True)
        qm_ref[...] = jnp.maximum(qm_ref[...], hit)

    t0 = t3_ref[0, 0]
    t1 = t3_ref[0, 1]
    t2 = t3_ref[0, 2]
    r0 = c1_ref[pl.ds(t0, 1), :]
    r1 = c1_ref[pl.ds(t1, 1), :]
    r2 = c1_ref[pl.ds(t2, 1), :]
    rows = jnp.concatenate([r0, r1, r2, r0, r0, r0, r0, r0], axis=0)
    heavy(rows.astype(jnp.float32), None)

    def outer(o, oc):
        blkmax = jnp.max(ip_ref[pl.ds(o * 256, 256), :])

        @pl.when(blkmax > _THRESH)
        def _scan_inner():
            def inner(s, ic):
                base = o * 256 + s * _KNN_R
                p8 = ip_ref[pl.ds(base, _KNN_R), :]

                @pl.when(jnp.max(p8) > _THRESH)
                def _do():
                    rows8 = c1_ref[pl.ds(base, _KNN_R), :].astype(jnp.float32)
                    heavy(rows8, p8 > _THRESH)

                return ic

            lax.fori_loop(0, 256 // _KNN_R, inner, 0)

        return oc

    lax.fori_loop(0, _N // 256, outer, 0)


def _branch2_body(x_ref, qm_ref, mW_ref, mb_ref, aW_ref, ab_ref, gW_ref, gb_ref,
                  cW_ref, cb_ref, kW_ref, kb_ref, m1_ref, fW1_ref, fW2_ref, fb_ref,
                  l3_ref, l2_ref, stat_ref, macc_ref):
    i = pl.program_id(0)

    @pl.when(i == 0)
    def _init():
        stat_ref[0] = _NEG
        stat_ref[1] = 0.0
        macc_ref[...] = jnp.zeros_like(macc_ref)

    x = x_ref[...]
    h = jnp.maximum(
        jnp.dot(x, mW_ref[...], preferred_element_type=jnp.float32)
        + mb_ref[...], 0.0)
    a = jnp.tanh(
        jnp.dot(h, aW_ref[...], preferred_element_type=jnp.float32)
        + ab_ref[...])
    g = jax.nn.sigmoid(
        jnp.dot(h, gW_ref[...], preferred_element_type=jnp.float32)
        + gb_ref[...])
    A = jnp.sum(a * g * cW_ref[...], axis=1, keepdims=True) + cb_ref[0, 0]
    mask = qm_ref[...] > 0.0
    Am = jnp.where(mask, A, _NEG)

    bm = jnp.max(Am)
    m_old = stat_ref[0]
    d_old = stat_ref[1]
    m_new = jnp.maximum(m_old, bm)
    alpha = jnp.exp(m_old - m_new)
    w = jnp.where(mask, jnp.exp(Am - m_new), 0.0)
    stat_ref[0] = m_new
    stat_ref[1] = d_old * alpha + jnp.sum(w)
    contrib = lax.dot_general(w, h, (((0,), (0,)), ((), ())),
                              preferred_element_type=jnp.float32)
    macc_ref[...] = macc_ref[...] * alpha + contrib

    @pl.when(i == _NBLK - 1)
    def _fin():
        M2 = macc_ref[...] / stat_ref[1]
        l2_ref[...] = jnp.dot(M2, kW_ref[...],
                              preferred_element_type=jnp.float32) + kb_ref[...]
        l3_ref[...] = (jnp.dot(m1_ref[...], fW1_ref[...],
                               preferred_element_type=jnp.float32)
                       + jnp.dot(M2, fW2_ref[...],
                                 preferred_element_type=jnp.float32)
                       + fb_ref[...])


def _const_spec(shape):
    return pl.BlockSpec(shape, lambda i: (0,) * len(shape))


def kernel(x1, x2, coords1, coords2,
           b1_mW, b1_mb, b1_aW, b1_ab, b1_gW, b1_gb, b1_cW, b1_cb, b1_kW, b1_kb,
           b2_mW, b2_mb, b2_aW, b2_ab, b2_gW, b2_gb, b2_cW, b2_cb, b2_kW, b2_kb,
           fW, fb):
    f32 = jnp.float32

    ip, M1, l1, t3 = pl.pallas_call(
        _branch1_body,
        grid=(_NBLK,),
        in_specs=[
            pl.BlockSpec((_BLK, 1024), lambda i: (i, 0)),
            _const_spec((1024, 512)),
            _const_spec((1, 512)),
            _const_spec((512, 256)),
            _const_spec((1, 256)),
            _const_spec((512, 256)),
            _const_spec((1, 256)),
            _const_spec((1, 256)),
            pl.BlockSpec(memory_space=pltpu.SMEM),
            _const_spec((512, 2)),
            _const_spec((1, 2)),
        ],
        out_specs=[
            pl.BlockSpec((_BLK, 1), lambda i: (i, 0)),
            _const_spec((1, 512)),
            _const_spec((1, 2)),
            _const_spec((1, 8)),
        ],
        out_shape=[
            jax.ShapeDtypeStruct((_N, 1), f32),
            jax.ShapeDtypeStruct((1, 512), f32),
            jax.ShapeDtypeStruct((1, 2), f32),
            jax.ShapeDtypeStruct((1, 8), jnp.int32),
        ],
        scratch_shapes=[
            pltpu.SMEM((2,), f32),
            pltpu.VMEM((1, 512), f32),
            pltpu.SMEM((3,), f32),
            pltpu.SMEM((3,), jnp.int32),
        ],
    )(x1, b1_mW, b1_mb.reshape(1, 512), b1_aW, b1_ab.reshape(1, 256),
      b1_gW, b1_gb.reshape(1, 256), b1_cW.reshape(1, 256),
      b1_cb.reshape(1, 1), b1_kW, b1_kb.reshape(1, 2))

    c2 = coords2.astype(f32)

    qm = pl.pallas_call(
        _knn_body,
        in_specs=[
            pl.BlockSpec(memory_space=pltpu.SMEM),
            pl.BlockSpec(memory_space=pltpu.VMEM),
            pl.BlockSpec(memory_space=pltpu.VMEM),
            pl.BlockSpec(memory_space=pltpu.VMEM),
            pl.BlockSpec(memory_space=pltpu.VMEM),
        ],
        out_specs=pl.BlockSpec(memory_space=pltpu.VMEM),
        out_shape=jax.ShapeDtypeStruct((1, _N), f32),
    )(t3, ip, coords1,
      c2[:, 0].reshape(1, _N), c2[:, 1].reshape(1, _N))

    qmc = qm.reshape(_N, 1)

    l3, l2 = pl.pallas_call(
        _branch2_body,
        grid=(_NBLK,),
        in_specs=[
            pl.BlockSpec((_BLK, 1024), lambda i: (i, 0)),
            pl.BlockSpec((_BLK, 1), lambda i: (i, 0)),
            _const_spec((1024, 512)),
            _const_spec((1, 512)),
            _const_spec((512, 256)),
            _const_spec((1, 256)),
            _const_spec((512, 256)),
            _const_spec((1, 256)),
            _const_spec((1, 256)),
            pl.BlockSpec(memory_space=pltpu.SMEM),
            _const_spec((512, 2)),
            _const_spec((1, 2)),
            _const_spec((1, 512)),
            _const_spec((512, 2)),
            _const_spec((512, 2)),
            _const_spec((1, 2)),
        ],
        out_specs=[
            _const_spec((1, 2)),
            _const_spec((1, 2)),
        ],
        out_shape=[
            jax.ShapeDtypeStruct((1, 2), f32),
            jax.ShapeDtypeStruct((1, 2), f32),
        ],
        scratch_shapes=[
            pltpu.SMEM((2,), f32),
            pltpu.VMEM((1, 512), f32),
        ],
    )(x2, qmc, b2_mW, b2_mb.reshape(1, 512), b2_aW, b2_ab.reshape(1, 256),
      b2_gW, b2_gb.reshape(1, 256), b2_cW.reshape(1, 256),
      b2_cb.reshape(1, 1), b2_kW, b2_kb.reshape(1, 2), M1,
      fW[:512], fW[512:], fb.reshape(1, 2))

    return (l3, l1, l2)


# speedup vs baseline: 1.2634x; 1.2634x over previous
import jax
import jax.numpy as jnp
from jax import lax
from jax.experimental import pallas as pl
from jax.experimental.pallas import tpu as pltpu

_N = 16384
_BLK = 2048
_NBLK = _N // _BLK
_KNN_R = 8
_TOPK = 3
_NEIGHK = 24
_THRESH = 0.5
_NEG = -1e30


def _branch1_body(x_ref, mW_ref, mb_ref, aW_ref, ab_ref, gW_ref, gb_ref,
                  cW_ref, cb_ref, kW_ref, kb_ref,
                  ip_ref, m1_ref, l1_ref, t3_ref,
                  stat_ref, macc_ref, t3v_ref, t3i_ref):
    i = pl.program_id(0)

    @pl.when(i == 0)
    def _init():
        stat_ref[0] = _NEG
        stat_ref[1] = 0.0
        macc_ref[...] = jnp.zeros_like(macc_ref)
        t3v_ref[0] = _NEG
        t3v_ref[1] = _NEG
        t3v_ref[2] = _NEG
        t3i_ref[0] = 0
        t3i_ref[1] = 0
        t3i_ref[2] = 0

    x = x_ref[...]
    h = jnp.maximum(
        jnp.dot(x, mW_ref[...], preferred_element_type=jnp.float32)
        + mb_ref[...], 0.0)
    a = jnp.tanh(
        jnp.dot(h, aW_ref[...], preferred_element_type=jnp.float32)
        + ab_ref[...])
    g = jax.nn.sigmoid(
        jnp.dot(h, gW_ref[...], preferred_element_type=jnp.float32)
        + gb_ref[...])
    A = jnp.sum(a * g * cW_ref[...], axis=1, keepdims=True) + cb_ref[0, 0]
    Ar = A.T
    ilr = (jnp.dot(h, kW_ref[...], preferred_element_type=jnp.float32)
           + kb_ref[...]).T
    ilm = jnp.max(ilr, axis=0, keepdims=True)
    pe = jnp.exp(ilr - ilm)
    p1 = pe[1:2, :] / (pe[0:1, :] + pe[1:2, :])
    probs = jax.nn.sigmoid(Ar) * p1
    ip_ref[...] = probs.T

    bm = jnp.max(Ar)
    m_old = stat_ref[0]
    d_old = stat_ref[1]
    m_new = jnp.maximum(m_old, bm)
    alpha = jnp.exp(m_old - m_new)
    w = jnp.exp(Ar - m_new)
    stat_ref[0] = m_new
    stat_ref[1] = d_old * alpha + jnp.sum(w)
    contrib = lax.dot_general(w, h, (((1,), (0,)), ((), ())),
                              preferred_element_type=jnp.float32)
    macc_ref[...] = macc_ref[...] * alpha + contrib

    gidx = i * _BLK + lax.broadcasted_iota(jnp.int32, (1, _BLK), 1)
    pv = probs
    for _ in range(_TOPK):
        bv = jnp.max(pv)
        bi = jnp.min(jnp.where(pv == bv, gidx, _N))
        v0, v1, v2 = t3v_ref[0], t3v_ref[1], t3v_ref[2]
        i0, i1, i2 = t3i_ref[0], t3i_ref[1], t3i_ref[2]
        gt0 = bv > v0
        gt1 = bv > v1
        gt2 = bv > v2
        t3v_ref[0] = jnp.where(gt0, bv, v0)
        t3i_ref[0] = jnp.where(gt0, bi, i0)
        t3v_ref[1] = jnp.where(gt0, v0, jnp.where(gt1, bv, v1))
        t3i_ref[1] = jnp.where(gt0, i0, jnp.where(gt1, bi, i1))
        t3v_ref[2] = jnp.where(gt0, v1, jnp.where(gt1, v1, jnp.where(gt2, bv, v2)))
        t3i_ref[2] = jnp.where(gt0, i1, jnp.where(gt1, i1, jnp.where(gt2, bi, i2)))
        pv = jnp.where(gidx == bi, _NEG, pv)

    @pl.when(i == _NBLK - 1)
    def _fin():
        M1 = macc_ref[...] / stat_ref[1]
        m1_ref[...] = M1
        l1_ref[...] = jnp.dot(M1, kW_ref[...],
                              preferred_element_type=jnp.float32) + kb_ref[...]
        lanes = lax.broadcasted_iota(jnp.int32, (1, 8), 1)
        t0, t1, t2 = t3i_ref[0], t3i_ref[1], t3i_ref[2]
        t3_ref[...] = jnp.where(lanes == 0, t0,
                                jnp.where(lanes == 1, t1,
                                          jnp.where(lanes == 2, t2, 0)))


def _knn_body(t3_ref, ip_ref, c1_ref, c2_ref, qm_ref):
    qm_ref[...] = jnp.zeros_like(qm_ref)
    c2t = c2_ref[...].T.astype(jnp.float32)
    c2x = c2t[0:1, :]
    c2y = c2t[1:2, :]
    lanes = lax.broadcasted_iota(jnp.int32, (_KNN_R, _N), 1)

    def heavy(rows8, valid):
        rx = rows8[:, 0:1]
        ry = rows8[:, 1:2]
        dx = rx - c2x
        dy = ry - c2y
        d2 = dx * dx + dy * dy
        work = d2
        m = jnp.zeros((_KNN_R, 1), jnp.float32)
        am = jnp.zeros((_KNN_R, 1), jnp.int32)
        for t in range(_NEIGHK):
            m = jnp.min(work, axis=1, keepdims=True)
            am = jnp.min(jnp.where(work == m, lanes, _N), axis=1, keepdims=True)
            if t < _NEIGHK - 1:
                work = jnp.where(lanes == am, jnp.inf, work)
        member = (d2 < m) | ((d2 == m) & (lanes <= am))
        if valid is not None:
            member = member & valid
        hit = jnp.max(member.astype(jnp.float32), axis=0, keepdims=True)
        qm_ref[...] = jnp.maximum(qm_ref[...], hit)

    t0 = t3_ref[0, 0]
    t1 = t3_ref[0, 1]
    t2 = t3_ref[0, 2]
    r0 = c1_ref[pl.ds(t0, 1), :]
    r1 = c1_ref[pl.ds(t1, 1), :]
    r2 = c1_ref[pl.ds(t2, 1), :]
    rows = jnp.concatenate([r0, r1, r2, r0, r0, r0, r0, r0], axis=0)
    heavy(rows.astype(jnp.float32), None)

    def outer(o, oc):
        blkmax = jnp.max(ip_ref[pl.ds(o * 256, 256), :])

        @pl.when(blkmax > _THRESH)
        def _scan_inner():
            def inner(s, ic):
                base = o * 256 + s * _KNN_R
                p8 = ip_ref[pl.ds(base, _KNN_R), :]

                @pl.when(jnp.max(p8) > _THRESH)
                def _do():
                    rows8 = c1_ref[pl.ds(base, _KNN_R), :].astype(jnp.float32)
                    heavy(rows8, p8 > _THRESH)

                return ic

            lax.fori_loop(0, 256 // _KNN_R, inner, 0)

        return oc

    lax.fori_loop(0, _N // 256, outer, 0)


def _branch2_body(x_ref, qm_ref, mW_ref, mb_ref, aW_ref, ab_ref, gW_ref, gb_ref,
                  cW_ref, cb_ref, kW_ref, kb_ref, m1_ref, fW1_ref, fW2_ref, fb_ref,
                  l3_ref, l2_ref, stat_ref, macc_ref):
    i = pl.program_id(0)

    @pl.when(i == 0)
    def _init():
        stat_ref[0] = _NEG
        stat_ref[1] = 0.0
        macc_ref[...] = jnp.zeros_like(macc_ref)

    x = x_ref[...]
    h = jnp.maximum(
        jnp.dot(x, mW_ref[...], preferred_element_type=jnp.float32)
        + mb_ref[...], 0.0)
    a = jnp.tanh(
        jnp.dot(h, aW_ref[...], preferred_element_type=jnp.float32)
        + ab_ref[...])
    g = jax.nn.sigmoid(
        jnp.dot(h, gW_ref[...], preferred_element_type=jnp.float32)
        + gb_ref[...])
    A = jnp.sum(a * g * cW_ref[...], axis=1, keepdims=True) + cb_ref[0, 0]
    Ar = A.T
    mask = qm_ref[...] > 0.0
    Am = jnp.where(mask, Ar, _NEG)

    bm = jnp.max(Am)
    m_old = stat_ref[0]
    d_old = stat_ref[1]
    m_new = jnp.maximum(m_old, bm)
    alpha = jnp.exp(m_old - m_new)
    w = jnp.where(mask, jnp.exp(Am - m_new), 0.0)
    stat_ref[0] = m_new
    stat_ref[1] = d_old * alpha + jnp.sum(w)
    contrib = lax.dot_general(w, h, (((1,), (0,)), ((), ())),
                              preferred_element_type=jnp.float32)
    macc_ref[...] = macc_ref[...] * alpha + contrib

    @pl.when(i == _NBLK - 1)
    def _fin():
        M2 = macc_ref[...] / stat_ref[1]
        l2_ref[...] = jnp.dot(M2, kW_ref[...],
                              preferred_element_type=jnp.float32) + kb_ref[...]
        l3_ref[...] = (jnp.dot(m1_ref[...], fW1_ref[...],
                               preferred_element_type=jnp.float32)
                       + jnp.dot(M2, fW2_ref[...],
                                 preferred_element_type=jnp.float32)
                       + fb_ref[...])


def _const_spec(shape):
    return pl.BlockSpec(shape, lambda i: (0,) * len(shape))


def kernel(x1, x2, coords1, coords2,
           b1_mW, b1_mb, b1_aW, b1_ab, b1_gW, b1_gb, b1_cW, b1_cb, b1_kW, b1_kb,
           b2_mW, b2_mb, b2_aW, b2_ab, b2_gW, b2_gb, b2_cW, b2_cb, b2_kW, b2_kb,
           fW, fb):
    f32 = jnp.float32

    ip, M1, l1, t3 = pl.pallas_call(
        _branch1_body,
        grid=(_NBLK,),
        in_specs=[
            pl.BlockSpec((_BLK, 1024), lambda i: (i, 0)),
            _const_spec((1024, 512)),
            _const_spec((1, 512)),
            _const_spec((512, 256)),
            _const_spec((1, 256)),
            _const_spec((512, 256)),
            _const_spec((1, 256)),
            _const_spec((1, 256)),
            pl.BlockSpec(memory_space=pltpu.SMEM),
            _const_spec((512, 2)),
            _const_spec((1, 2)),
        ],
        out_specs=[
            pl.BlockSpec((_BLK, 1), lambda i: (i, 0)),
            _const_spec((1, 512)),
            _const_spec((1, 2)),
            _const_spec((1, 8)),
        ],
        out_shape=[
            jax.ShapeDtypeStruct((_N, 1), f32),
            jax.ShapeDtypeStruct((1, 512), f32),
            jax.ShapeDtypeStruct((1, 2), f32),
            jax.ShapeDtypeStruct((1, 8), jnp.int32),
        ],
        scratch_shapes=[
            pltpu.SMEM((2,), f32),
            pltpu.VMEM((1, 512), f32),
            pltpu.SMEM((3,), f32),
            pltpu.SMEM((3,), jnp.int32),
        ],
    )(x1, b1_mW, b1_mb.reshape(1, 512), b1_aW, b1_ab.reshape(1, 256),
      b1_gW, b1_gb.reshape(1, 256), b1_cW.reshape(1, 256),
      b1_cb.reshape(1, 1), b1_kW, b1_kb.reshape(1, 2))

    qm = pl.pallas_call(
        _knn_body,
        in_specs=[
            pl.BlockSpec(memory_space=pltpu.SMEM),
            pl.BlockSpec(memory_space=pltpu.VMEM),
            pl.BlockSpec(memory_space=pltpu.VMEM),
            pl.BlockSpec(memory_space=pltpu.VMEM),
        ],
        out_specs=pl.BlockSpec(memory_space=pltpu.VMEM),
        out_shape=jax.ShapeDtypeStruct((1, _N), f32),
    )(t3, ip, coords1, coords2)

    l3, l2 = pl.pallas_call(
        _branch2_body,
        grid=(_NBLK,),
        in_specs=[
            pl.BlockSpec((_BLK, 1024), lambda i: (i, 0)),
            pl.BlockSpec((1, _BLK), lambda i: (0, i)),
            _const_spec((1024, 512)),
            _const_spec((1, 512)),
            _const_spec((512, 256)),
            _const_spec((1, 256)),
            _const_spec((512, 256)),
            _const_spec((1, 256)),
            _const_spec((1, 256)),
            pl.BlockSpec(memory_space=pltpu.SMEM),
            _const_spec((512, 2)),
            _const_spec((1, 2)),
            _const_spec((1, 512)),
            _const_spec((512, 2)),
            _const_spec((512, 2)),
            _const_spec((1, 2)),
        ],
        out_specs=[
            _const_spec((1, 2)),
            _const_spec((1, 2)),
        ],
        out_shape=[
            jax.ShapeDtypeStruct((1, 2), f32),
            jax.ShapeDtypeStruct((1, 2), f32),
        ],
        scratch_shapes=[
            pltpu.SMEM((2,), f32),
            pltpu.VMEM((1, 512), f32),
        ],
    )(x2, qm, b2_mW, b2_mb.reshape(1, 512), b2_aW, b2_ab.reshape(1, 256),
      b2_gW, b2_gb.reshape(1, 256), b2_cW.reshape(1, 256),
      b2_cb.reshape(1, 1), b2_kW, b2_kb.reshape(1, 2), M1,
      fW[:512], fW[512:], fb.reshape(1, 2))

    return (l3, l1, l2)
